# baseline (speedup 1.0000x reference)
import functools

import jax
import jax.numpy as jnp
from jax import lax
from jax.experimental import pallas as pl
from jax.experimental.pallas import tpu as pltpu
from jax.experimental.pallas import tpu_sc as plsc

EMBED_DIM = 32
NUM_CORES = 2
NUM_SUBCORES = 16
NUM_WORKERS = NUM_CORES * NUM_SUBCORES
CHUNK = 128


def _sc_gather(table, idx3d):
    n_ch = idx3d.shape[1]
    rows_per_w = n_ch * CHUNK
    total_rows = NUM_WORKERS * rows_per_w
    mesh = plsc.VectorSubcoreMesh(core_axis_name="c", subcore_axis_name="s")

    @functools.partial(
        pl.kernel,
        mesh=mesh,
        compiler_params=pltpu.CompilerParams(use_tc_tiling_on_sc=False),
        out_type=jax.ShapeDtypeStruct((total_rows, EMBED_DIM), jnp.float32),
        scratch_types=[
            pltpu.VMEM((n_ch, CHUNK), jnp.int32),
            pltpu.VMEM((CHUNK, EMBED_DIM), jnp.float32),
            pltpu.SemaphoreType.DMA,
        ],
    )
    def k(idx_hbm, table_hbm, out_hbm, idx_v, rows_v, sem):
        wid = lax.axis_index("s") * NUM_CORES + lax.axis_index("c")
        base_row = wid * rows_per_w
        pltpu.sync_copy(idx_hbm.at[wid], idx_v)

        def body(j, carry):
            pltpu.async_copy(table_hbm.at[idx_v.at[j]], rows_v, sem).wait()
            pltpu.sync_copy(rows_v, out_hbm.at[pl.ds(base_row + j * CHUNK, CHUNK)])
            return carry

        lax.fori_loop(0, n_ch, body, 0)

    return k(idx3d, table)


def kernel(x, table):
    idx = x.astype(jnp.int32).reshape(-1)
    idx3d = idx.reshape(NUM_WORKERS, -1, CHUNK)
    out = _sc_gather(table, idx3d)
    return out.reshape(x.shape + (EMBED_DIM,))


# speedup vs baseline: 1.0469x; 1.0469x over previous
import functools

import jax
import jax.numpy as jnp
from jax import lax
from jax.experimental import pallas as pl
from jax.experimental.pallas import tpu as pltpu
from jax.experimental.pallas import tpu_sc as plsc

EMBED_DIM = 32
NUM_CORES = 2
NUM_SUBCORES = 16
NUM_WORKERS = NUM_CORES * NUM_SUBCORES
CHUNK = 800


def _sc_gather(table, idx3d):
    n_ch = idx3d.shape[1]
    rows_per_w = n_ch * CHUNK
    total_rows = NUM_WORKERS * rows_per_w
    mesh = plsc.VectorSubcoreMesh(core_axis_name="c", subcore_axis_name="s")

    @functools.partial(
        pl.kernel,
        mesh=mesh,
        compiler_params=pltpu.CompilerParams(use_tc_tiling_on_sc=False),
        out_type=jax.ShapeDtypeStruct((total_rows, EMBED_DIM), jnp.float32),
        scratch_types=[
            pltpu.VMEM((n_ch, CHUNK), jnp.int32),
            pltpu.VMEM((CHUNK, EMBED_DIM), jnp.float32),
            pltpu.VMEM((CHUNK, EMBED_DIM), jnp.float32),
            pltpu.SemaphoreType.DMA,
            pltpu.SemaphoreType.DMA,
        ],
    )
    def k(idx_hbm, table_hbm, out_hbm, idx_v, rows0, rows1, sem0, sem1):
        wid = lax.axis_index("s") * NUM_CORES + lax.axis_index("c")
        base_row = wid * rows_per_w
        pltpu.sync_copy(idx_hbm.at[wid], idx_v)

        def fire(j, buf, sem):
            pltpu.async_copy(table_hbm.at[idx_v.at[j]], buf, sem)

        def drain(j, buf, sem):
            pltpu.make_async_copy(table_hbm.at[idx_v.at[j]], buf, sem).wait()

        def scatter(j, buf):
            pltpu.sync_copy(buf, out_hbm.at[pl.ds(base_row + j * CHUNK, CHUNK)])

        fire(0, rows0, sem0)

        def body(p, carry):
            j0 = 2 * p
            fire(j0 + 1, rows1, sem1)
            drain(j0, rows0, sem0)
            scatter(j0, rows0)

            @pl.when(j0 + 2 < n_ch)
            def _():
                fire(j0 + 2, rows0, sem0)

            drain(j0 + 1, rows1, sem1)
            scatter(j0 + 1, rows1)
            return carry

        lax.fori_loop(0, n_ch // 2, body, 0)

    return k(idx3d, table)


def kernel(x, table):
    idx = x.astype(jnp.int32).reshape(-1)
    idx3d = idx.reshape(NUM_WORKERS, -1, CHUNK)
    out = _sc_gather(table, idx3d)
    return out.reshape(x.shape + (EMBED_DIM,))
